# baseline (speedup 1.0000x reference)
import functools
import math

import jax
import jax.numpy as jnp
from jax import lax
from jax.experimental import pallas as pl
from jax.experimental.pallas import tpu as pltpu
from jax.experimental.pallas import tpu_sc as plsc

D_MODEL = 64
LANES = 16
NUM_CORES = 2
NUM_SUBCORES = 16
NUM_WORKERS = NUM_CORES * NUM_SUBCORES
SCALE = math.sqrt(D_MODEL)

CHUNK = 1024


def _make_kernel(n_idx):
    assert n_idx % (NUM_WORKERS * CHUNK) == 0
    per_worker = n_idx // NUM_WORKERS
    n_chunks = per_worker // CHUNK
    mesh = plsc.VectorSubcoreMesh(core_axis_name="c", subcore_axis_name="s")

    @functools.partial(
        pl.kernel,
        mesh=mesh,
        out_type=jax.ShapeDtypeStruct((n_idx, D_MODEL), jnp.float32),
        scratch_types=[
            pltpu.VMEM((CHUNK,), jnp.int32),
            pltpu.VMEM((CHUNK, D_MODEL), jnp.float32),
            pltpu.SemaphoreType.DMA,
        ],
        compiler_params=pltpu.CompilerParams(use_tc_tiling_on_sc=False),
    )
    def emb_kernel(x_hbm, table_hbm, out_hbm, idx_v, rows_v, gsem):
        wid = lax.axis_index("s") * NUM_CORES + lax.axis_index("c")
        base = wid * per_worker

        def chunk_body(i, carry):
            off = base + i * CHUNK
            pltpu.sync_copy(x_hbm.at[pl.ds(off, CHUNK)], idx_v)
            pltpu.async_copy(table_hbm.at[idx_v], rows_v, gsem).wait()

            def scale_body(r, c):
                for j in range(D_MODEL // LANES):
                    s = pl.ds(j * LANES, LANES)
                    rows_v[r, s] = rows_v[r, s] * SCALE
                return c

            lax.fori_loop(0, CHUNK, scale_body, 0)
            pltpu.sync_copy(rows_v, out_hbm.at[pl.ds(off, CHUNK)])
            return carry

        lax.fori_loop(0, n_chunks, chunk_body, 0)

    return emb_kernel


@jax.jit
def kernel(x, table):
    b, l = x.shape
    x_flat = x.reshape((b * l,)).astype(jnp.int32)
    out = _make_kernel(b * l)(x_flat, table)
    return out.reshape((b, l, D_MODEL))


# speedup vs baseline: 1.1078x; 1.1078x over previous
import functools
import math

import jax
import jax.numpy as jnp
from jax import lax
from jax.experimental import pallas as pl
from jax.experimental.pallas import tpu as pltpu
from jax.experimental.pallas import tpu_sc as plsc

D_MODEL = 64
LANES = 16
NUM_CORES = 2
NUM_SUBCORES = 16
NUM_WORKERS = NUM_CORES * NUM_SUBCORES
SCALE = math.sqrt(D_MODEL)

CHUNK = 512
NSLOT = 3


def _make_kernel(n_idx):
    assert n_idx % (NUM_WORKERS * CHUNK) == 0
    per_worker = n_idx // NUM_WORKERS
    n_chunks = per_worker // CHUNK
    mesh = plsc.VectorSubcoreMesh(core_axis_name="c", subcore_axis_name="s")

    scratch = (
        [pltpu.VMEM((CHUNK,), jnp.int32) for _ in range(NSLOT)]
        + [pltpu.VMEM((CHUNK, D_MODEL), jnp.float32) for _ in range(NSLOT)]
        + [pltpu.SemaphoreType.DMA for _ in range(3 * NSLOT)]
    )

    @functools.partial(
        pl.kernel,
        mesh=mesh,
        out_type=jax.ShapeDtypeStruct((n_idx, D_MODEL), jnp.float32),
        scratch_types=scratch,
        compiler_params=pltpu.CompilerParams(use_tc_tiling_on_sc=False),
    )
    def emb_kernel(x_hbm, table_hbm, out_hbm, *s):
        idx = s[0:NSLOT]
        rows = s[NSLOT:2 * NSLOT]
        isem = s[2 * NSLOT:3 * NSLOT]
        gsem = s[3 * NSLOT:4 * NSLOT]
        osem = s[4 * NSLOT:5 * NSLOT]

        wid = lax.axis_index("s") * NUM_CORES + lax.axis_index("c")
        base = wid * per_worker

        def idx_start(i):
            pltpu.async_copy(
                x_hbm.at[pl.ds(base + i * CHUNK, CHUNK)], idx[i % NSLOT],
                isem[i % NSLOT])

        def idx_wait(i):
            pltpu.make_async_copy(
                x_hbm.at[pl.ds(base + i * CHUNK, CHUNK)], idx[i % NSLOT],
                isem[i % NSLOT]).wait()

        def gather_start(i):
            pltpu.async_copy(
                table_hbm.at[idx[i % NSLOT]], rows[i % NSLOT], gsem[i % NSLOT])

        def gather_wait(i):
            pltpu.make_async_copy(
                table_hbm.at[idx[i % NSLOT]], rows[i % NSLOT],
                gsem[i % NSLOT]).wait()

        def store_start(i):
            pltpu.async_copy(
                rows[i % NSLOT], out_hbm.at[pl.ds(base + i * CHUNK, CHUNK)],
                osem[i % NSLOT])

        def store_wait(i):
            pltpu.make_async_copy(
                rows[i % NSLOT], out_hbm.at[pl.ds(base + i * CHUNK, CHUNK)],
                osem[i % NSLOT]).wait()

        def scale(i):
            r = rows[i % NSLOT]

            def scale_body(t, c):
                row = t * 4
                for u in range(4):
                    for j in range(D_MODEL // LANES):
                        sl = pl.ds(j * LANES, LANES)
                        r[row + u, sl] = r[row + u, sl] * SCALE
                return c

            lax.fori_loop(0, CHUNK // 4, scale_body, 0)

        for i in range(min(NSLOT, n_chunks)):
            idx_start(i)
        idx_wait(0)
        gather_start(0)

        for i in range(n_chunks):
            gather_wait(i)
            if i + NSLOT < n_chunks:
                idx_start(i + NSLOT)
            if i + 1 < n_chunks:
                if i - (NSLOT - 1) >= 0:
                    store_wait(i - (NSLOT - 1))
                idx_wait(i + 1)
                gather_start(i + 1)
            scale(i)
            store_start(i)

        for i in range(max(0, n_chunks - NSLOT), n_chunks):
            store_wait(i)

    return emb_kernel


@jax.jit
def kernel(x, table):
    b, l = x.shape
    x_flat = x.reshape((b * l,)).astype(jnp.int32)
    out = _make_kernel(b * l)(x_flat, table)
    return out.reshape((b, l, D_MODEL))
